# baseline (speedup 1.0000x reference)
import functools

import jax
import jax.numpy as jnp
from jax import lax
from jax.experimental import pallas as pl
from jax.experimental.pallas import tpu as pltpu
from jax.experimental.pallas import tpu_sc as plsc

N = 10000
D = 128
E = 320000

NC = 2
NS = 16
NW = NC * NS

K = 128
CPW = 80
HALF = CPW // 2
EPW = CPW * K
E_PAD = NW * EPW
TRASH = N
NP = 10112
RPT = NP // NS

_mesh = lambda: plsc.VectorSubcoreMesh(core_axis_name="c", subcore_axis_name="s")


def _deg_body(col_hbm, out_hbm, cidx, hist):
    c = lax.axis_index("c")
    s = lax.axis_index("s")
    wid = s * NC + c
    pltpu.sync_copy(col_hbm.at[pl.ds(wid * EPW, EPW)], cidx)
    zeros16 = jnp.zeros((16,), jnp.float32)

    def zstep(i, _):
        hist[pl.ds(i * 16, 16)] = zeros16
        return _

    lax.fori_loop(0, NP // 16, zstep, None)
    ones16 = jnp.ones((16,), jnp.float32)

    def step(j, _):
        ids = cidx[pl.ds(j * 16, 16)]
        plsc.addupdate_scatter(hist, [ids], ones16)
        return _

    lax.fori_loop(0, EPW // 16, step, None)
    pltpu.sync_copy(hist, out_hbm.at[pl.ds(wid * NP, NP)])


_deg_kernel = functools.partial(
    pl.kernel,
    out_type=jax.ShapeDtypeStruct((NW * NP,), jnp.float32),
    mesh=_mesh(),
    scratch_types=[
        pltpu.VMEM((EPW,), jnp.int32),
        pltpu.VMEM((NP,), jnp.float32),
    ],
    compiler_params=pltpu.CompilerParams(needs_layout_passes=False),
)(_deg_body)


def _edge_body(hs_hbm, row_hbm, col_hbm, zeros_hbm, out_hbm,
               ridx, cidx, rows0, rows1, acc_sh, sem0, sem1):
    c = lax.axis_index("c")
    s = lax.axis_index("s")
    wid = s * NC + c
    pltpu.sync_copy(zeros_hbm, acc_sh.at[pl.ds(s * RPT, RPT)])
    plsc.subcore_barrier()

    bufs = (rows0, rows1)
    sems = (sem0, sem1)

    for h in range(2):
        base = wid * CPW + h * HALF
        pltpu.sync_copy(row_hbm.at[pl.ds(base, HALF)], ridx)
        pltpu.sync_copy(col_hbm.at[pl.ds(base, HALF)], cidx)
        pltpu.async_copy(hs_hbm.at[ridx.at[0]], rows0, sem0)
        pltpu.async_copy(hs_hbm.at[ridx.at[1]], rows1, sem1)

        def pair(j, _):
            for b in range(2):
                i = 2 * j + b
                buf, sem = bufs[b], sems[b]
                pltpu.make_async_copy(hs_hbm.at[ridx.at[i]], buf, sem).wait()
                pltpu.sync_copy(buf, acc_sh.at[cidx.at[i]], add=True)

                @pl.when(i + 2 < HALF)
                def _():
                    pltpu.async_copy(hs_hbm.at[ridx.at[i + 2]], buf, sem)
            return _

        lax.fori_loop(0, HALF // 2, pair, None)
    plsc.subcore_barrier()
    pltpu.sync_copy(acc_sh.at[pl.ds(s * RPT, RPT)],
                    out_hbm.at[pl.ds(c * NP + s * RPT, RPT)])


_edge_kernel = functools.partial(
    pl.kernel,
    out_type=jax.ShapeDtypeStruct((NC * NP, D), jnp.float32),
    mesh=_mesh(),
    scratch_types=[
        pltpu.VMEM((HALF, K), jnp.int32),
        pltpu.VMEM((HALF, K), jnp.int32),
        pltpu.VMEM((K, D), jnp.float32),
        pltpu.VMEM((K, D), jnp.float32),
        pltpu.VMEM_SHARED((NP, D), jnp.float32),
        pltpu.SemaphoreType.DMA,
        pltpu.SemaphoreType.DMA,
    ],
)(_edge_body)


BR = 2000


def _linear_body(x_ref, w_ref, degp_ref, hs_ref, dinv_ref):
    deg = jnp.sum(degp_ref[...], axis=1, keepdims=True) + 1.0
    dinv = lax.rsqrt(deg)
    h = jnp.dot(x_ref[...], w_ref[...], preferred_element_type=jnp.float32)
    hs_ref[...] = h * dinv
    dinv_ref[...] = dinv


def _linear_tc(x, w, degp):
    return pl.pallas_call(
        _linear_body,
        grid=(N // BR,),
        in_specs=[
            pl.BlockSpec((BR, D), lambda i: (i, 0)),
            pl.BlockSpec((D, D), lambda i: (0, 0)),
            pl.BlockSpec((BR, NW), lambda i: (i, 0)),
        ],
        out_specs=[
            pl.BlockSpec((BR, D), lambda i: (i, 0)),
            pl.BlockSpec((BR, 1), lambda i: (i, 0)),
        ],
        out_shape=[
            jax.ShapeDtypeStruct((N, D), jnp.float32),
            jax.ShapeDtypeStruct((N, 1), jnp.float32),
        ],
    )(x, w, degp)


def _finish_body(accp_ref, hs_ref, dinv_ref, b_ref, out_ref):
    acc = accp_ref[0] + accp_ref[1] + hs_ref[...]
    out_ref[...] = jnp.maximum(acc * dinv_ref[...] + b_ref[...], 0.0)


def _finish_tc(accp, hs, dinv, b2):
    return pl.pallas_call(
        _finish_body,
        grid=(N // BR,),
        in_specs=[
            pl.BlockSpec((NC, BR, D), lambda i: (0, i, 0)),
            pl.BlockSpec((BR, D), lambda i: (i, 0)),
            pl.BlockSpec((BR, 1), lambda i: (i, 0)),
            pl.BlockSpec((1, D), lambda i: (0, 0)),
        ],
        out_specs=pl.BlockSpec((BR, D), lambda i: (i, 0)),
        out_shape=jax.ShapeDtypeStruct((N, D), jnp.float32),
    )(accp, hs, dinv, b2)


@jax.jit
def _run(A, X, W, b):
    A = A.astype(jnp.int32)
    pad = E_PAD - E
    row_p = jnp.concatenate([A[0], jnp.zeros((pad,), jnp.int32)])
    col_p = jnp.concatenate([A[1], jnp.full((pad,), TRASH, jnp.int32)])

    zerosD = jnp.zeros((RPT, D), jnp.float32)

    row3 = row_p.reshape(NW * CPW, K)
    col3 = col_p.reshape(NW * CPW, K)

    deg_flat = _deg_kernel(col_p)
    degp = deg_flat.reshape(NW, NP).T[:N]

    hs, dinv = _linear_tc(X, W, degp)

    acc_flat = _edge_kernel(hs, row3, col3, zerosD)
    accp = acc_flat.reshape(NC, NP, D)[:, :N, :]

    return _finish_tc(accp, hs, dinv, b.reshape(1, D))


def kernel(A, X, W, b):
    return _run(A, X, W, b)


# speedup vs baseline: 3.2950x; 3.2950x over previous
import functools

import jax
import jax.numpy as jnp
from jax import lax
from jax.experimental import pallas as pl
from jax.experimental.pallas import tpu as pltpu
from jax.experimental.pallas import tpu_sc as plsc

N = 10000
D = 128
E = 320000

NC = 2
NS = 16
NW = NC * NS

K = 125
CPW = 80
HALF = CPW // 2
EPW = CPW * K
NP = 10112
RPT = NP // NS

_mesh = lambda: plsc.VectorSubcoreMesh(core_axis_name="c", subcore_axis_name="s")


def _deg_body(col_hbm, out_hbm, cidx, hist):
    c = lax.axis_index("c")
    s = lax.axis_index("s")
    wid = s * NC + c
    pltpu.sync_copy(col_hbm.at[pl.ds(wid * EPW, EPW)], cidx)
    zeros16 = jnp.zeros((16,), jnp.float32)

    def zstep(i, _):
        hist[pl.ds(i * 16, 16)] = zeros16
        return _

    lax.fori_loop(0, NP // 16, zstep, None)
    ones16 = jnp.ones((16,), jnp.float32)

    def step(j, _):
        ids = cidx[pl.ds(j * 16, 16)]
        plsc.addupdate_scatter(hist, [ids], ones16)
        return _

    lax.fori_loop(0, EPW // 16, step, None)
    pltpu.sync_copy(hist, out_hbm.at[pl.ds(wid * NP, NP)])


_deg_kernel = functools.partial(
    pl.kernel,
    out_type=jax.ShapeDtypeStruct((NW * NP,), jnp.float32),
    mesh=_mesh(),
    scratch_types=[
        pltpu.VMEM((EPW,), jnp.int32),
        pltpu.VMEM((NP,), jnp.float32),
    ],
    compiler_params=pltpu.CompilerParams(needs_layout_passes=False),
)(_deg_body)


def _edge_body(hs_hbm, row_hbm, col_hbm, zeros_hbm, out_hbm,
               ridx, cidx, rows0, rows1, acc_sh, sem0, sem1):
    c = lax.axis_index("c")
    s = lax.axis_index("s")
    wid = s * NC + c
    pltpu.sync_copy(zeros_hbm, acc_sh.at[pl.ds(s * RPT, RPT)])
    plsc.subcore_barrier()

    bufs = (rows0, rows1)
    sems = (sem0, sem1)

    for h in range(2):
        base = wid * CPW + h * HALF
        pltpu.sync_copy(row_hbm.at[pl.ds(base, HALF)], ridx)
        pltpu.sync_copy(col_hbm.at[pl.ds(base, HALF)], cidx)
        pltpu.async_copy(hs_hbm.at[ridx.at[0]], rows0, sem0)
        pltpu.async_copy(hs_hbm.at[ridx.at[1]], rows1, sem1)

        def pair(j, _):
            for b in range(2):
                i = 2 * j + b
                buf, sem = bufs[b], sems[b]
                pltpu.make_async_copy(hs_hbm.at[ridx.at[i]], buf, sem).wait()
                pltpu.sync_copy(buf, acc_sh.at[cidx.at[i]], add=True)

                @pl.when(i + 2 < HALF)
                def _():
                    pltpu.async_copy(hs_hbm.at[ridx.at[i + 2]], buf, sem)
            return _

        lax.fori_loop(0, HALF // 2, pair, None)
    plsc.subcore_barrier()
    pltpu.sync_copy(acc_sh.at[pl.ds(s * RPT, RPT)],
                    out_hbm.at[pl.ds(c * NP + s * RPT, RPT)])


_edge_kernel = functools.partial(
    pl.kernel,
    out_type=jax.ShapeDtypeStruct((NC * NP, D), jnp.float32),
    mesh=_mesh(),
    scratch_types=[
        pltpu.VMEM((HALF, K), jnp.int32),
        pltpu.VMEM((HALF, K), jnp.int32),
        pltpu.VMEM((K, D), jnp.float32),
        pltpu.VMEM((K, D), jnp.float32),
        pltpu.VMEM_SHARED((NP, D), jnp.float32),
        pltpu.SemaphoreType.DMA,
        pltpu.SemaphoreType.DMA,
    ],
)(_edge_body)


BR = 2000


def _linear_body(x_ref, w_ref, degp_ref, hs_ref, dinv_ref):
    deg = jnp.sum(degp_ref[...], axis=1, keepdims=True) + 1.0
    dinv = lax.rsqrt(deg)
    h = jnp.dot(x_ref[...], w_ref[...], preferred_element_type=jnp.float32)
    hs_ref[...] = h * dinv
    dinv_ref[...] = dinv


def _linear_tc(x, w, degp):
    return pl.pallas_call(
        _linear_body,
        grid=(N // BR,),
        in_specs=[
            pl.BlockSpec((BR, D), lambda i: (i, 0)),
            pl.BlockSpec((D, D), lambda i: (0, 0)),
            pl.BlockSpec((BR, NW), lambda i: (i, 0)),
        ],
        out_specs=[
            pl.BlockSpec((BR, D), lambda i: (i, 0)),
            pl.BlockSpec((BR, 1), lambda i: (i, 0)),
        ],
        out_shape=[
            jax.ShapeDtypeStruct((N, D), jnp.float32),
            jax.ShapeDtypeStruct((N, 1), jnp.float32),
        ],
    )(x, w, degp)


def _finish_body(accp_ref, hs_ref, dinv_ref, b_ref, out_ref):
    acc = accp_ref[0] + accp_ref[1] + hs_ref[...]
    out_ref[...] = jnp.maximum(acc * dinv_ref[...] + b_ref[...], 0.0)


def _finish_tc(accp, hs, dinv, b2):
    return pl.pallas_call(
        _finish_body,
        grid=(N // BR,),
        in_specs=[
            pl.BlockSpec((NC, BR, D), lambda i: (0, i, 0)),
            pl.BlockSpec((BR, D), lambda i: (i, 0)),
            pl.BlockSpec((BR, 1), lambda i: (i, 0)),
            pl.BlockSpec((1, D), lambda i: (0, 0)),
        ],
        out_specs=pl.BlockSpec((BR, D), lambda i: (i, 0)),
        out_shape=jax.ShapeDtypeStruct((N, D), jnp.float32),
    )(accp, hs, dinv, b2)


@jax.jit
def _run(A, X, W, b):
    A = A.astype(jnp.int32)
    row_p = A[0]
    col_p = A[1]

    zerosD = jnp.zeros((RPT, D), jnp.float32)

    row3 = row_p.reshape(NW * CPW, K)
    col3 = col_p.reshape(NW * CPW, K)

    deg_flat = _deg_kernel(col_p)
    degp = deg_flat.reshape(NW, NP).T[:N]

    hs, dinv = _linear_tc(X, W, degp)

    acc_flat = _edge_kernel(hs, row3, col3, zerosD)
    accp = acc_flat.reshape(NC, NP, D)[:, :N, :]

    return _finish_tc(accp, hs, dinv, b.reshape(1, D))


def kernel(A, X, W, b):
    return _run(A, X, W, b)


# speedup vs baseline: 3.3285x; 1.0101x over previous
import functools

import jax
import jax.numpy as jnp
from jax import lax
from jax.experimental import pallas as pl
from jax.experimental.pallas import tpu as pltpu
from jax.experimental.pallas import tpu_sc as plsc

N = 10000
D = 128
E = 320000

NC = 2
NS = 16
NW = NC * NS

K = 125
CPW = 80
HALF = CPW // 2
EPW = CPW * K
NP = 10112
RPT = NP // NS

_mesh = lambda: plsc.VectorSubcoreMesh(core_axis_name="c", subcore_axis_name="s")


def _deg_body(col_hbm, out_hbm, cidx, hist):
    c = lax.axis_index("c")
    s = lax.axis_index("s")
    wid = s * NC + c
    pltpu.sync_copy(col_hbm.at[pl.ds(wid * EPW, EPW)], cidx)
    zeros16 = jnp.zeros((16,), jnp.float32)

    def zstep(i, _):
        hist[pl.ds(i * 16, 16)] = zeros16
        return _

    lax.fori_loop(0, NP // 16, zstep, None)
    ones16 = jnp.ones((16,), jnp.float32)

    def step(j, _):
        ids = cidx[pl.ds(j * 16, 16)]
        plsc.addupdate_scatter(hist, [ids], ones16)
        return _

    lax.fori_loop(0, EPW // 16, step, None)
    pltpu.sync_copy(hist, out_hbm.at[pl.ds(wid * NP, NP)])


_deg_kernel = functools.partial(
    pl.kernel,
    out_type=jax.ShapeDtypeStruct((NW * NP,), jnp.float32),
    mesh=_mesh(),
    scratch_types=[
        pltpu.VMEM((EPW,), jnp.int32),
        pltpu.VMEM((NP,), jnp.float32),
    ],
    compiler_params=pltpu.CompilerParams(needs_layout_passes=False),
)(_deg_body)


def _edge_body(hs_hbm, row_hbm, col_hbm, zeros_hbm, out_hbm,
               ridx, cidx, rows0, rows1, acc_sh, sem0, sem1):
    c = lax.axis_index("c")
    s = lax.axis_index("s")
    wid = s * NC + c
    pltpu.sync_copy(zeros_hbm, acc_sh.at[pl.ds(s * RPT, RPT)])
    plsc.subcore_barrier()

    bufs = (rows0, rows1)
    sems = (sem0, sem1)

    for h in range(2):
        base = wid * CPW + h * HALF
        pltpu.sync_copy(row_hbm.at[pl.ds(base, HALF)], ridx)
        pltpu.sync_copy(col_hbm.at[pl.ds(base, HALF)], cidx)
        pltpu.async_copy(hs_hbm.at[ridx.at[0]], rows0, sem0)
        pltpu.async_copy(hs_hbm.at[ridx.at[1]], rows1, sem1)

        def pair(j, _):
            for b in range(2):
                i = 2 * j + b
                buf, sem = bufs[b], sems[b]
                pltpu.make_async_copy(hs_hbm.at[ridx.at[i]], buf, sem).wait()
                pltpu.sync_copy(buf, acc_sh.at[cidx.at[i]], add=True)

                @pl.when(i + 2 < HALF)
                def _():
                    pltpu.async_copy(hs_hbm.at[ridx.at[i + 2]], buf, sem)
            return _

        lax.fori_loop(0, HALF // 2, pair, None)
    plsc.subcore_barrier()
    pltpu.sync_copy(acc_sh.at[pl.ds(s * RPT, RPT)],
                    out_hbm.at[pl.ds(c * NP + s * RPT, RPT)])


_edge_kernel = functools.partial(
    pl.kernel,
    out_type=jax.ShapeDtypeStruct((NC * NP, D), jnp.float32),
    mesh=_mesh(),
    scratch_types=[
        pltpu.VMEM((HALF, K), jnp.int32),
        pltpu.VMEM((HALF, K), jnp.int32),
        pltpu.VMEM((K, D), jnp.float32),
        pltpu.VMEM((K, D), jnp.float32),
        pltpu.VMEM_SHARED((NP, D), jnp.float32),
        pltpu.SemaphoreType.DMA,
        pltpu.SemaphoreType.DMA,
    ],
)(_edge_body)


BR = 2000


def _matmul_body(x_ref, w_ref, h_ref):
    h_ref[...] = jnp.dot(x_ref[...], w_ref[...],
                         preferred_element_type=jnp.float32)


def _matmul_tc(x, w):
    return pl.pallas_call(
        _matmul_body,
        grid=(N // BR,),
        in_specs=[
            pl.BlockSpec((BR, D), lambda i: (i, 0)),
            pl.BlockSpec((D, D), lambda i: (0, 0)),
        ],
        out_specs=pl.BlockSpec((BR, D), lambda i: (i, 0)),
        out_shape=jax.ShapeDtypeStruct((N, D), jnp.float32),
    )(x, w)


def _scale_body(h_ref, degp_ref, hs_ref, dinv_ref):
    deg = jnp.sum(degp_ref[...], axis=1, keepdims=True) + 1.0
    dinv = lax.rsqrt(deg)
    hs_ref[...] = h_ref[...] * dinv
    dinv_ref[...] = dinv


def _scale_tc(h, degp):
    return pl.pallas_call(
        _scale_body,
        grid=(N // BR,),
        in_specs=[
            pl.BlockSpec((BR, D), lambda i: (i, 0)),
            pl.BlockSpec((BR, NW), lambda i: (i, 0)),
        ],
        out_specs=[
            pl.BlockSpec((BR, D), lambda i: (i, 0)),
            pl.BlockSpec((BR, 1), lambda i: (i, 0)),
        ],
        out_shape=[
            jax.ShapeDtypeStruct((N, D), jnp.float32),
            jax.ShapeDtypeStruct((N, 1), jnp.float32),
        ],
    )(h, degp)


def _finish_body(accp_ref, hs_ref, dinv_ref, b_ref, out_ref):
    acc = accp_ref[0] + accp_ref[1] + hs_ref[...]
    out_ref[...] = jnp.maximum(acc * dinv_ref[...] + b_ref[...], 0.0)


def _finish_tc(accp, hs, dinv, b2):
    return pl.pallas_call(
        _finish_body,
        grid=(N // BR,),
        in_specs=[
            pl.BlockSpec((NC, BR, D), lambda i: (0, i, 0)),
            pl.BlockSpec((BR, D), lambda i: (i, 0)),
            pl.BlockSpec((BR, 1), lambda i: (i, 0)),
            pl.BlockSpec((1, D), lambda i: (0, 0)),
        ],
        out_specs=pl.BlockSpec((BR, D), lambda i: (i, 0)),
        out_shape=jax.ShapeDtypeStruct((N, D), jnp.float32),
    )(accp, hs, dinv, b2)


@jax.jit
def _run(A, X, W, b):
    A = A.astype(jnp.int32)
    row_p = A[0]
    col_p = A[1]

    zerosD = jnp.zeros((RPT, D), jnp.float32)

    row3 = row_p.reshape(NW * CPW, K)
    col3 = col_p.reshape(NW * CPW, K)

    deg_flat = _deg_kernel(col_p)
    degp = deg_flat.reshape(NW, NP).T[:N]

    h = _matmul_tc(X, W)
    hs, dinv = _scale_tc(h, degp)

    acc_flat = _edge_kernel(hs, row3, col3, zerosD)
    accp = acc_flat.reshape(NC, NP, D)

    return _finish_tc(accp, hs, dinv, b.reshape(1, D))


def kernel(A, X, W, b):
    return _run(A, X, W, b)


# speedup vs baseline: 3.4252x; 1.0291x over previous
import functools

import jax
import jax.numpy as jnp
from jax import lax
from jax.experimental import pallas as pl
from jax.experimental.pallas import tpu as pltpu
from jax.experimental.pallas import tpu_sc as plsc

N = 10000
D = 128
E = 320000

NC = 2
NS = 16
NW = NC * NS

K = 125
CPW = 80
HALF = CPW // 2
EPW = CPW * K
NP = 10112
RPT = NP // NS

_mesh = lambda: plsc.VectorSubcoreMesh(core_axis_name="c", subcore_axis_name="s")


def _deg_body(col_hbm, out_hbm, cidx, hist):
    c = lax.axis_index("c")
    s = lax.axis_index("s")
    wid = s * NC + c
    pltpu.sync_copy(col_hbm.at[pl.ds(wid * EPW, EPW)], cidx)
    zeros16 = jnp.zeros((16,), jnp.float32)

    def zstep(i, _):
        hist[pl.ds(i * 16, 16)] = zeros16
        return _

    lax.fori_loop(0, NP // 16, zstep, None)
    ones16 = jnp.ones((16,), jnp.float32)

    def step(j, _):
        ids = cidx[pl.ds(j * 16, 16)]
        plsc.addupdate_scatter(hist, [ids], ones16)
        return _

    lax.fori_loop(0, EPW // 16, step, None)
    pltpu.sync_copy(hist, out_hbm.at[pl.ds(wid * NP, NP)])


_deg_kernel = functools.partial(
    pl.kernel,
    out_type=jax.ShapeDtypeStruct((NW * NP,), jnp.float32),
    mesh=_mesh(),
    scratch_types=[
        pltpu.VMEM((EPW,), jnp.int32),
        pltpu.VMEM((NP,), jnp.float32),
    ],
    compiler_params=pltpu.CompilerParams(needs_layout_passes=False),
)(_deg_body)


def _edge_body(hs_hbm, row_hbm, col_hbm, zeros_hbm, out_hbm,
               ridx, cidx, rows0, rows1, acc_sh, sem0, sem1):
    c = lax.axis_index("c")
    s = lax.axis_index("s")
    wid = s * NC + c
    pltpu.sync_copy(zeros_hbm, acc_sh.at[pl.ds(s * RPT, RPT)])
    plsc.subcore_barrier()

    bufs = (rows0, rows1)
    sems = (sem0, sem1)

    for h in range(2):
        base = wid * CPW + h * HALF
        pltpu.sync_copy(row_hbm.at[pl.ds(base, HALF)], ridx)
        pltpu.sync_copy(col_hbm.at[pl.ds(base, HALF)], cidx)
        pltpu.async_copy(hs_hbm.at[ridx.at[0]], rows0, sem0)
        pltpu.async_copy(hs_hbm.at[ridx.at[1]], rows1, sem1)

        def pair(j, _):
            for b in range(2):
                i = 2 * j + b
                buf, sem = bufs[b], sems[b]
                pltpu.make_async_copy(hs_hbm.at[ridx.at[i]], buf, sem).wait()
                pltpu.sync_copy(buf, acc_sh.at[cidx.at[i]], add=True)

                @pl.when(i + 2 < HALF)
                def _():
                    pltpu.async_copy(hs_hbm.at[ridx.at[i + 2]], buf, sem)
            return _

        lax.fori_loop(0, HALF // 2, pair, None)
    plsc.subcore_barrier()
    pltpu.sync_copy(acc_sh.at[pl.ds(s * RPT, RPT)],
                    out_hbm.at[pl.ds(c * NP + s * RPT, RPT)])


_edge_kernel = functools.partial(
    pl.kernel,
    out_type=jax.ShapeDtypeStruct((NC * NP, D), jnp.float32),
    mesh=_mesh(),
    scratch_types=[
        pltpu.VMEM((HALF, K), jnp.int32),
        pltpu.VMEM((HALF, K), jnp.int32),
        pltpu.VMEM((K, D), jnp.float32),
        pltpu.VMEM((K, D), jnp.float32),
        pltpu.VMEM_SHARED((NP, D), jnp.float32),
        pltpu.SemaphoreType.DMA,
        pltpu.SemaphoreType.DMA,
    ],
)(_edge_body)


BR = 2000


def _linear_body(x_ref, w_ref, degp_ref, hs_ref, dinv_ref):
    deg = jnp.sum(degp_ref[...], axis=1, keepdims=True) + 1.0
    dinv = lax.rsqrt(deg)
    h = jnp.dot(x_ref[...], w_ref[...], preferred_element_type=jnp.float32)
    hs_ref[...] = h * dinv
    dinv_ref[...] = dinv


def _linear_tc(x, w, degp):
    return pl.pallas_call(
        _linear_body,
        grid=(N // BR,),
        in_specs=[
            pl.BlockSpec((BR, D), lambda i: (i, 0)),
            pl.BlockSpec((D, D), lambda i: (0, 0)),
            pl.BlockSpec((BR, NW), lambda i: (i, 0)),
        ],
        out_specs=[
            pl.BlockSpec((BR, D), lambda i: (i, 0)),
            pl.BlockSpec((BR, 1), lambda i: (i, 0)),
        ],
        out_shape=[
            jax.ShapeDtypeStruct((N, D), jnp.float32),
            jax.ShapeDtypeStruct((N, 1), jnp.float32),
        ],
    )(x, w, degp)


def _finish_body(accp_ref, hs_ref, dinv_ref, b_ref, out_ref):
    acc = accp_ref[0] + accp_ref[1] + hs_ref[...]
    out_ref[...] = jnp.maximum(acc * dinv_ref[...] + b_ref[...], 0.0)


def _finish_tc(accp, hs, dinv, b2):
    return pl.pallas_call(
        _finish_body,
        grid=(N // BR,),
        in_specs=[
            pl.BlockSpec((NC, BR, D), lambda i: (0, i, 0)),
            pl.BlockSpec((BR, D), lambda i: (i, 0)),
            pl.BlockSpec((BR, 1), lambda i: (i, 0)),
            pl.BlockSpec((1, D), lambda i: (0, 0)),
        ],
        out_specs=pl.BlockSpec((BR, D), lambda i: (i, 0)),
        out_shape=jax.ShapeDtypeStruct((N, D), jnp.float32),
    )(accp, hs, dinv, b2)


@jax.jit
def _run(A, X, W, b):
    A = A.astype(jnp.int32)
    row_p = A[0]
    col_p = A[1]

    zerosD = jnp.zeros((RPT, D), jnp.float32)

    row3 = row_p.reshape(NW * CPW, K)
    col3 = col_p.reshape(NW * CPW, K)

    deg_flat = _deg_kernel(col_p)
    degp = deg_flat.reshape(NW, NP).T[:N]

    hs, dinv = _linear_tc(X, W, degp)

    acc_flat = _edge_kernel(hs, row3, col3, zerosD)
    accp = acc_flat.reshape(NC, NP, D)

    return _finish_tc(accp, hs, dinv, b.reshape(1, D))


def kernel(A, X, W, b):
    return _run(A, X, W, b)
